# baseline (speedup 1.0000x reference)
import functools

import jax
import jax.numpy as jnp
from jax import lax
from jax.experimental import pallas as pl
from jax.experimental.pallas import tpu as pltpu
from jax.experimental.pallas import tpu_sc as plsc

N = 1048576
EMBED = 16
TABLE = 2097152
MASK = TABLE - 1

NC = 2
NS = 16
NW = NC * NS
B_PER_W = N // NW
C = 2048
NCHUNK = B_PER_W // C
R = 128
G = C // R

import numpy as np

P2 = np.uint32(2654435761).astype(np.int64).astype(np.int32)
P3 = np.int32(805459861)


def _cell(x):
    t = (x * 0.5 + 0.5) * 128.0
    c = t.astype(jnp.int32)
    return jnp.clip(c, 0, 127)


def _body(xyz_hbm, table_hbm, out_hbm, xyz_v, idx_v, rows_v, sem_g):
    wid = lax.axis_index("s") * NC + lax.axis_index("c")
    base = wid * B_PER_W

    def chunk(j, carry):
        pbase = base + j * C
        pltpu.sync_copy(xyz_hbm.at[pl.ds(pbase * 3, C * 3)], xyz_v)

        def hvec(i, carry2):
            xi = (lax.iota(jnp.int32, 16) + i * 16) * 3
            x = plsc.load_gather(xyz_v, [xi])
            y = plsc.load_gather(xyz_v, [xi + 1])
            z = plsc.load_gather(xyz_v, [xi + 2])
            h = _cell(x) ^ (_cell(y) * P2) ^ (_cell(z) * P3)
            idx_v[pl.ds(i * 16, 16)] = h & MASK
            return carry2

        lax.fori_loop(0, C // 16, hvec, 0)

        cps = [
            pltpu.async_copy(
                table_hbm.at[idx_v.at[pl.ds(r * R, R)]],
                rows_v.at[pl.ds(r * R, R)],
                sem_g,
            )
            for r in range(G)
        ]
        for cp in cps:
            cp.wait()
        pltpu.sync_copy(rows_v, out_hbm.at[pl.ds(pbase, C)])
        return carry

    lax.fori_loop(0, NCHUNK, chunk, 0)


@functools.partial(jax.jit, static_argnames=())
def kernel(xyz, table):
    assert xyz.shape == (N, 3) and table.shape == (TABLE, EMBED)
    lookup = pl.kernel(
        _body,
        out_type=jax.ShapeDtypeStruct((N, EMBED), jnp.float32),
        mesh=plsc.VectorSubcoreMesh(core_axis_name="c", subcore_axis_name="s"),
        scratch_types=[
            pltpu.VMEM((C * 3,), jnp.float32),
            pltpu.VMEM((C,), jnp.int32),
            pltpu.VMEM((C, EMBED), jnp.float32),
            pltpu.SemaphoreType.DMA,
        ],
        compiler_params=pltpu.CompilerParams(
            needs_layout_passes=False, use_tc_tiling_on_sc=False
        ),
    )
    return lookup(xyz.reshape(-1), table)


# speedup vs baseline: 1.6263x; 1.6263x over previous
import functools

import numpy as np
import jax
import jax.numpy as jnp
from jax import lax
from jax.experimental import pallas as pl
from jax.experimental.pallas import tpu as pltpu
from jax.experimental.pallas import tpu_sc as plsc

N = 1048576
EMBED = 16
TABLE = 2097152
MASK = TABLE - 1

NC = 2
NS = 16
NW = NC * NS
B_PER_W = N // NW

TBLK = TABLE // 128
TBLK_W = TBLK // NW
HC = 1024
NHC = B_PER_W // HC

GC = 1024
NGC = B_PER_W // GC
R = 128
G_PER_C = GC // R
PLANE = (N // 128) * 1024

P2 = np.uint32(2654435761).astype(np.int64).astype(np.int32)
P3 = np.int32(805459861)


def _cell(x):
    t = (x * 0.5 + 0.5) * 128.0
    c = t.astype(jnp.int32)
    return jnp.clip(c, 0, 127)


def _prep_body(tt_hbm, xt_hbm, trf_hbm, idx_hbm, tbuf, rbuf, xv, yv, iv, hsem):
    wid = lax.axis_index("s") * NC + lax.axis_index("c")

    def tblock(b, carry):
        blk = wid * TBLK_W + b
        pltpu.sync_copy(tt_hbm.at[:, pl.ds(blk * 128, 128)], tbuf)
        lanes = lax.iota(jnp.int32, 16)
        zeros = jnp.zeros((16,), jnp.int32)

        def trow(u, carry2):
            for k in range(8):
                r = u * 8 + k
                v = plsc.load_gather(tbuf, [lanes, zeros + r])
                rbuf[pl.ds(r * 16, 16)] = v
            return carry2

        lax.fori_loop(0, 16, trow, 0)
        pltpu.sync_copy(rbuf, trf_hbm.at[pl.ds(blk * 2048, 2048)])
        return carry

    lax.fori_loop(0, TBLK_W, tblock, 0)

    lanes16 = lax.iota(jnp.int32, 16)
    xbufs = (xv, yv)
    zeros16 = jnp.zeros((16,), jnp.int32)

    def hchunk(j, carry):
        pbase = wid * B_PER_W + j * HC
        cp0 = pltpu.async_copy(
            xt_hbm.at[:, pl.ds(pbase, 128)], xbufs[0], hsem)
        cps = {0: cp0}
        for k in range(HC // 128):
            if k + 1 < HC // 128:
                cps[k + 1] = pltpu.async_copy(
                    xt_hbm.at[:, pl.ds(pbase + (k + 1) * 128, 128)],
                    xbufs[(k + 1) % 2], hsem)
            cps[k].wait()
            buf = xbufs[k % 2]
            for g in range(8):
                sl = pl.ds(g * 16, 16)
                x = plsc.load_gather(buf, [zeros16, g * 16 + lanes16])
                y = plsc.load_gather(buf, [zeros16 + 1, g * 16 + lanes16])
                z = plsc.load_gather(buf, [zeros16 + 2, g * 16 + lanes16])
                h = _cell(x) ^ (_cell(y) * P2) ^ (_cell(z) * P3)
                iv[pl.ds(k * 128 + g * 16, 16)] = h & MASK
        pltpu.sync_copy(iv, idx_hbm.at[pl.ds(pbase, HC)])
        return carry

    lax.fori_loop(0, NHC, hchunk, 0)


def _gather_body(trf_hbm, idx_hbm, of_hbm, iv, rows, st0, st1, sem_g):
    wid = lax.axis_index("s") * NC + lax.axis_index("c")
    lanes = lax.iota(jnp.int32, 16)

    def chunk(j, carry):
        pbase = wid * B_PER_W + j * GC
        pltpu.sync_copy(idx_hbm.at[pl.ds(pbase, GC)], iv)
        cps = [
            pltpu.async_copy(
                trf_hbm.at[iv.at[pl.ds(r * R, R)]],
                rows.at[pl.ds(r * R, R)],
                sem_g,
            )
            for r in range(G_PER_C)
        ]
        for cp in cps:
            cp.wait()

        def tpose(m, carry2):
            t = m // 8
            ci = m % 8
            for cb, st in ((0, st0), (1, st1)):
                for g in range(8):
                    pidx = t * 128 + g * 16 + lanes
                    cidx = jnp.zeros((16,), jnp.int32) + (cb * 8 + ci)
                    v = plsc.load_gather(rows, [pidx, cidx])
                    st[pl.ds(t * 1024 + ci * 128 + g * 16, 16)] = v
            return carry2

        lax.fori_loop(0, (GC // 128) * 8, tpose, 0)

        tile0 = (pbase // 128) * 1024
        pltpu.sync_copy(st0, of_hbm.at[pl.ds(tile0, (GC // 128) * 1024)])
        pltpu.sync_copy(st1, of_hbm.at[pl.ds(PLANE + tile0, (GC // 128) * 1024)])
        return carry

    lax.fori_loop(0, NGC, chunk, 0)


def kernel(xyz, table):
    assert xyz.shape == (N, 3) and table.shape == (TABLE, EMBED)
    prep = pl.kernel(
        _prep_body,
        out_type=(
            jax.ShapeDtypeStruct((TABLE * EMBED,), jnp.float32),
            jax.ShapeDtypeStruct((N,), jnp.int32),
        ),
        mesh=plsc.VectorSubcoreMesh(core_axis_name="c", subcore_axis_name="s"),
        scratch_types=[
            pltpu.VMEM((16, 128), jnp.float32),
            pltpu.VMEM((2048,), jnp.float32),
            pltpu.VMEM((3, 128), jnp.float32),
            pltpu.VMEM((3, 128), jnp.float32),
            pltpu.VMEM((HC,), jnp.int32),
            pltpu.SemaphoreType.DMA,
        ],
        compiler_params=pltpu.CompilerParams(
            needs_layout_passes=False, use_tc_tiling_on_sc=True
        ),
    )
    gather = pl.kernel(
        _gather_body,
        out_type=jax.ShapeDtypeStruct((N * EMBED,), jnp.float32),
        mesh=plsc.VectorSubcoreMesh(core_axis_name="c", subcore_axis_name="s"),
        scratch_types=[
            pltpu.VMEM((GC,), jnp.int32),
            pltpu.VMEM((GC, EMBED), jnp.float32),
            pltpu.VMEM(((GC // 128) * 1024,), jnp.float32),
            pltpu.VMEM(((GC // 128) * 1024,), jnp.float32),
            pltpu.SemaphoreType.DMA,
        ],
        compiler_params=pltpu.CompilerParams(
            needs_layout_passes=False, use_tc_tiling_on_sc=False
        ),
    )
    trf, idx = prep(table.T, xyz.T)
    of = gather(trf.reshape(TABLE, EMBED), idx)
    return of.reshape(2, N // 128, 8, 128).transpose(1, 3, 0, 2).reshape(N, EMBED)


# speedup vs baseline: 2.2926x; 1.4097x over previous
import functools

import numpy as np
import jax
import jax.numpy as jnp
from jax import lax
from jax.experimental import pallas as pl
from jax.experimental.pallas import tpu as pltpu
from jax.experimental.pallas import tpu_sc as plsc

N = 1048576
EMBED = 16
TABLE = 2097152
MASK = TABLE - 1

NC = 2
NS = 16
NW = NC * NS
B_PER_W = N // NW

TBLK_W = (TABLE // 128) // NW
HC = 1024
NHC = B_PER_W // HC

GC = 1024
NGC = B_PER_W // GC
R = 128
G_PER_C = GC // R
ST = (GC // 128) * 1024
PLANE = (N // 128) * 1024

P2 = np.uint32(2654435761).astype(np.int64).astype(np.int32)
P3 = np.int32(805459861)


def _cell(x):
    t = (x * 0.5 + 0.5) * 128.0
    c = t.astype(jnp.int32)
    return jnp.clip(c, 0, 127)


def _prep_body(tt_hbm, xt_hbm, trf_hbm, idx_hbm,
               tb0, tb1, rb0, rb1, xv, yv, iv,
               isem0, isem1, osem0, osem1, hsem):
    wid = lax.axis_index("s") * NC + lax.axis_index("c")
    lanes = lax.iota(jnp.int32, 16)
    zeros = jnp.zeros((16,), jnp.int32)
    tbufs = (tb0, tb1)
    rbufs = (rb0, rb1)
    isems = (isem0, isem1)
    osems = (osem0, osem1)
    base = wid * TBLK_W

    def in_args(b, par):
        return tt_hbm.at[:, pl.ds((base + b) * 128, 128)], tbufs[par]

    def out_args(b, par):
        return rbufs[par], trf_hbm.at[pl.ds((base + b) * 2048, 2048)]

    def transpose_block(par):
        tb, rb = tbufs[par], rbufs[par]

        def trow(u, c2):
            for k in range(8):
                r = u * 8 + k
                rb[pl.ds(r * 16, 16)] = plsc.load_gather(tb, [lanes, zeros + r])
            return c2

        lax.fori_loop(0, 16, trow, 0)

    pltpu.async_copy(*in_args(0, 0), isems[0])
    pltpu.async_copy(*in_args(1, 1), isems[1])
    for par in (0, 1):
        pltpu.make_async_copy(*in_args(par, par), isems[par]).wait()
        transpose_block(par)
        pltpu.async_copy(*out_args(par, par), osems[par])
        pltpu.async_copy(*in_args(par + 2, par), isems[par])

    def steady(jj, carry):
        for par in (0, 1):
            b = jj * 2 + par
            pltpu.make_async_copy(*in_args(b, par), isems[par]).wait()
            pltpu.make_async_copy(*out_args(b, par), osems[par]).wait()
            transpose_block(par)
            pltpu.async_copy(*out_args(b, par), osems[par])

            @pl.when(b + 2 < TBLK_W)
            def _():
                pltpu.async_copy(*in_args(b + 2, par), isems[par])

        return carry

    lax.fori_loop(1, TBLK_W // 2, steady, 0)
    for par in (0, 1):
        pltpu.make_async_copy(*out_args(0, par), osems[par]).wait()

    xbufs = (xv, yv)

    def hchunk(j, carry):
        pbase = wid * B_PER_W + j * HC
        cps = {0: pltpu.async_copy(
            xt_hbm.at[:, pl.ds(pbase, 128)], xbufs[0], hsem)}
        for k in range(HC // 128):
            if k + 1 < HC // 128:
                cps[k + 1] = pltpu.async_copy(
                    xt_hbm.at[:, pl.ds(pbase + (k + 1) * 128, 128)],
                    xbufs[(k + 1) % 2], hsem)
            cps[k].wait()
            buf = xbufs[k % 2]
            for g in range(8):
                pidx = g * 16 + lanes
                x = plsc.load_gather(buf, [zeros, pidx])
                y = plsc.load_gather(buf, [zeros + 1, pidx])
                z = plsc.load_gather(buf, [zeros + 2, pidx])
                h = _cell(x) ^ (_cell(y) * P2) ^ (_cell(z) * P3)
                iv[pl.ds(k * 128 + g * 16, 16)] = h & MASK
        pltpu.sync_copy(iv, idx_hbm.at[pl.ds(pbase, HC)])
        return carry

    lax.fori_loop(0, NHC, hchunk, 0)


def _gather_body(trf_hbm, idx_hbm, of_hbm,
                 iv0, iv1, rows0, rows1, s0a, s0b, s1a, s1b,
                 isem0, isem1, gsem0, gsem1, osem0, osem1):
    wid = lax.axis_index("s") * NC + lax.axis_index("c")
    lanes = lax.iota(jnp.int32, 16)
    zeros = jnp.zeros((16,), jnp.int32)
    ivs = (iv0, iv1)
    rowss = (rows0, rows1)
    st0s = (s0a, s0b)
    st1s = (s1a, s1b)
    isems = (isem0, isem1)
    gsems = (gsem0, gsem1)
    osems = (osem0, osem1)
    pb = wid * B_PER_W

    def idx_args(j, par):
        return idx_hbm.at[pl.ds(pb + j * GC, GC)], ivs[par]

    def fire_gathers(par):
        for r in range(G_PER_C):
            pltpu.async_copy(
                trf_hbm.at[ivs[par].at[pl.ds(r * R, R)]],
                rowss[par].at[pl.ds(r * R, R)],
                gsems[par],
            )

    def drain_gathers(par):
        pltpu.make_async_copy(
            trf_hbm.at[ivs[par]], rowss[par], gsems[par]).wait()

    def out_args(j, par, cb):
        tile0 = ((pb + j * GC) // 128) * 1024
        st = st0s[par] if cb == 0 else st1s[par]
        return st, of_hbm.at[pl.ds(cb * PLANE + tile0, ST)]

    def fire_outs(j, par):
        pltpu.async_copy(*out_args(j, par, 0), osems[par])
        pltpu.async_copy(*out_args(j, par, 1), osems[par])

    def drain_outs(par):
        pltpu.make_async_copy(*out_args(0, par, 0), osems[par]).wait()
        pltpu.make_async_copy(*out_args(0, par, 1), osems[par]).wait()

    def transpose_chunk(par):
        rows, st0, st1 = rowss[par], st0s[par], st1s[par]

        def tpose(m, c2):
            t = m // 8
            ci = m % 8
            for cb, st in ((0, st0), (1, st1)):
                for g in range(8):
                    v = plsc.load_gather(
                        rows, [t * 128 + g * 16 + lanes, zeros + (cb * 8 + ci)])
                    st[pl.ds(t * 1024 + ci * 128 + g * 16, 16)] = v
            return c2

        lax.fori_loop(0, (GC // 128) * 8, tpose, 0)

    pltpu.async_copy(*idx_args(0, 0), isems[0])
    pltpu.make_async_copy(*idx_args(0, 0), isems[0]).wait()
    fire_gathers(0)
    pltpu.async_copy(*idx_args(1, 1), isems[1])

    pltpu.make_async_copy(*idx_args(1, 1), isems[1]).wait()
    fire_gathers(1)
    drain_gathers(0)
    transpose_chunk(0)
    fire_outs(0, 0)
    pltpu.async_copy(*idx_args(2, 0), isems[0])

    pltpu.make_async_copy(*idx_args(2, 0), isems[0]).wait()
    fire_gathers(0)
    drain_gathers(1)
    transpose_chunk(1)
    fire_outs(1, 1)
    pltpu.async_copy(*idx_args(3, 1), isems[1])

    pltpu.make_async_copy(*idx_args(3, 1), isems[1]).wait()
    fire_gathers(1)
    drain_gathers(0)
    drain_outs(0)
    transpose_chunk(0)
    fire_outs(2, 0)
    pltpu.async_copy(*idx_args(4, 0), isems[0])

    def steady(jj, carry):
        for par in (0, 1):
            j = jj * 2 + par
            pltpu.make_async_copy(*idx_args(j, par), isems[par]).wait()
            fire_gathers(par)
            drain_gathers(1 - par)
            drain_outs(1 - par)
            transpose_chunk(1 - par)
            fire_outs(j - 1, 1 - par)

            @pl.when(j + 1 < NGC)
            def _():
                pltpu.async_copy(*idx_args(j + 1, 1 - par), isems[1 - par])

        return carry

    lax.fori_loop(2, NGC // 2, steady, 0)

    drain_gathers(1)
    drain_outs(1)
    transpose_chunk(1)
    fire_outs(NGC - 1, 1)
    drain_outs(0)
    drain_outs(1)


def kernel(xyz, table):
    assert xyz.shape == (N, 3) and table.shape == (TABLE, EMBED)
    prep = pl.kernel(
        _prep_body,
        out_type=(
            jax.ShapeDtypeStruct((TABLE * EMBED,), jnp.float32),
            jax.ShapeDtypeStruct((N,), jnp.int32),
        ),
        mesh=plsc.VectorSubcoreMesh(core_axis_name="c", subcore_axis_name="s"),
        scratch_types=[
            pltpu.VMEM((16, 128), jnp.float32),
            pltpu.VMEM((16, 128), jnp.float32),
            pltpu.VMEM((2048,), jnp.float32),
            pltpu.VMEM((2048,), jnp.float32),
            pltpu.VMEM((3, 128), jnp.float32),
            pltpu.VMEM((3, 128), jnp.float32),
            pltpu.VMEM((HC,), jnp.int32),
            pltpu.SemaphoreType.DMA,
            pltpu.SemaphoreType.DMA,
            pltpu.SemaphoreType.DMA,
            pltpu.SemaphoreType.DMA,
            pltpu.SemaphoreType.DMA,
        ],
        compiler_params=pltpu.CompilerParams(
            needs_layout_passes=False, use_tc_tiling_on_sc=True
        ),
    )
    gather = pl.kernel(
        _gather_body,
        out_type=jax.ShapeDtypeStruct((N * EMBED,), jnp.float32),
        mesh=plsc.VectorSubcoreMesh(core_axis_name="c", subcore_axis_name="s"),
        scratch_types=[
            pltpu.VMEM((GC,), jnp.int32),
            pltpu.VMEM((GC,), jnp.int32),
            pltpu.VMEM((GC, EMBED), jnp.float32),
            pltpu.VMEM((GC, EMBED), jnp.float32),
            pltpu.VMEM((ST,), jnp.float32),
            pltpu.VMEM((ST,), jnp.float32),
            pltpu.VMEM((ST,), jnp.float32),
            pltpu.VMEM((ST,), jnp.float32),
            pltpu.SemaphoreType.DMA,
            pltpu.SemaphoreType.DMA,
            pltpu.SemaphoreType.DMA,
            pltpu.SemaphoreType.DMA,
            pltpu.SemaphoreType.DMA,
            pltpu.SemaphoreType.DMA,
        ],
        compiler_params=pltpu.CompilerParams(
            needs_layout_passes=False, use_tc_tiling_on_sc=False
        ),
    )
    trf, idx = prep(table.T, xyz.T)
    of = gather(trf.reshape(TABLE, EMBED), idx)
    return of.reshape(2, N // 128, 8, 128).transpose(1, 3, 0, 2).reshape(N, EMBED)
